# Initial kernel scaffold; baseline (speedup 1.0000x reference)
import jax
import jax.numpy as jnp
from jax.experimental import pallas as pl


def kernel(x_acoustic, x_word, edge_sim_tic, edge_sim_w, edge_related_to, W1_tic, b1_tic, W1_w, b1_w, W1_rel, b1_rel, W2_tic, b2_tic, W2_w, b2_w, W2_rel, b2_rel):
    raise NotImplementedError("write your pallas kernel here")


# speedup vs baseline: 2.7029x; 2.7029x over previous
import functools

import jax
import jax.numpy as jnp
from jax import lax
from jax.experimental import pallas as pl
from jax.experimental.pallas import tpu as pltpu
from jax.experimental.pallas import tpu_sc as plsc

N = 10000
NPAD = 10240
GARBAGE = 10000
E = 160000
EPAD = 163840
D = 256
DH = 128
H = 512
O = 256
BM = 1024
NHIST = 6


def _hist_body(idx_hbm, zh_hbm, out_hbm, ibuf, hist, sem):
    c = lax.axis_index("c")
    s = lax.axis_index("s")
    wid = c * 16 + s
    ones = jnp.full((16,), 1.0, jnp.float32)
    for a in range(NHIST):
        pltpu.sync_copy(zh_hbm, hist.at[pl.ds(a * NPAD, NPAD)])
    for a in range(NHIST):
        base = a * EPAD + wid * 5120

        def chunk(r, base=base, a=a):
            pltpu.sync_copy(idx_hbm.at[pl.ds(base + r * 1024, 1024)], ibuf)
            for m in range(64):
                iv = ibuf[pl.ds(m * 16, 16)] + a * NPAD
                plsc.addupdate_scatter(hist, [iv], ones)

        pl.loop(0, 5)(chunk)
    pltpu.sync_copy(hist, out_hbm.at[pl.ds(wid * (NHIST * NPAD), NHIST * NPAD)])


def _make_hist():
    mesh = plsc.VectorSubcoreMesh(core_axis_name="c", subcore_axis_name="s")
    return functools.partial(
        pl.kernel,
        mesh=mesh,
        compiler_params=pltpu.CompilerParams(needs_layout_passes=False),
        out_type=jax.ShapeDtypeStruct((32 * NHIST * NPAD,), jnp.float32),
        scratch_types=[
            pltpu.VMEM((1024,), jnp.int32),
            pltpu.VMEM((NHIST * NPAD,), jnp.float32),
            pltpu.SemaphoreType.DMA,
        ],
    )(_hist_body)


def _agg_body(vlo_hbm, vhi_hbm, src_hbm, dst_hbm, z_hbm, olo_hbm, ohi_hbm,
              sidx, didx, rows, acc, sem):
    c = lax.axis_index("c")
    s = lax.axis_index("s")
    pltpu.sync_copy(z_hbm, acc.at[pl.ds(s * 640, 640)])
    plsc.subcore_barrier()

    def edge_loop(v_hbm):
        def chunk(g):
            row0 = s * 80 + g * 8
            pltpu.sync_copy(src_hbm.at[pl.ds(row0, 8)], sidx)
            pltpu.sync_copy(dst_hbm.at[pl.ds(row0, 8)], didx)
            for j in range(8):
                pltpu.async_copy(v_hbm.at[sidx.at[j]], rows, sem).wait()
                pltpu.sync_copy(rows, acc.at[didx.at[j]], add=True)

        pl.loop(0, 10)(chunk)

    @pl.when(c == 0)
    def _():
        edge_loop(vlo_hbm)

    @pl.when(c == 1)
    def _():
        edge_loop(vhi_hbm)

    plsc.subcore_barrier()

    @pl.when(c == 0)
    def _():
        pltpu.sync_copy(acc.at[pl.ds(s * 640, 640)], olo_hbm.at[pl.ds(s * 640, 640)])

    @pl.when(c == 1)
    def _():
        pltpu.sync_copy(acc.at[pl.ds(s * 640, 640)], ohi_hbm.at[pl.ds(s * 640, 640)])


def _make_agg():
    mesh = plsc.VectorSubcoreMesh(core_axis_name="c", subcore_axis_name="s")
    return functools.partial(
        pl.kernel,
        mesh=mesh,
        compiler_params=pltpu.CompilerParams(needs_layout_passes=False),
        out_type=[
            jax.ShapeDtypeStruct((NPAD, DH), jnp.float32),
            jax.ShapeDtypeStruct((NPAD, DH), jnp.float32),
        ],
        scratch_types=[
            pltpu.VMEM((8, 128), jnp.int32),
            pltpu.VMEM((8, 128), jnp.int32),
            pltpu.VMEM((128, DH), jnp.float32),
            pltpu.VMEM_SHARED((NPAD, DH), jnp.float32),
            pltpu.SemaphoreType.DMA,
        ],
    )(_agg_body)


def _tca_body(p_ref, xa_ref, xw_ref, sc_ref, xtl, xth, xwl, xwh, xrl, xrh):
    deg = jnp.sum(p_ref[...], axis=0)
    scl = lax.rsqrt(jnp.maximum(deg, 1.0))
    sc_ref[...] = scl
    xst = xa_ref[...] * scl[0][:, None]
    xsw = xw_ref[...] * scl[2][:, None]
    xsr = xa_ref[...] * scl[4][:, None]
    xtl[...] = xst[:, :DH]
    xth[...] = xst[:, DH:]
    xwl[...] = xsw[:, :DH]
    xwh[...] = xsw[:, DH:]
    xrl[...] = xsr[:, :DH]
    xrh[...] = xsr[:, DH:]


def _tca(p, xa_p, xw_p):
    g = NPAD // BM
    return pl.pallas_call(
        _tca_body,
        grid=(g,),
        in_specs=[
            pl.BlockSpec((32, NHIST, BM), lambda i: (0, 0, i)),
            pl.BlockSpec((BM, D), lambda i: (i, 0)),
            pl.BlockSpec((BM, D), lambda i: (i, 0)),
        ],
        out_specs=[pl.BlockSpec((NHIST, BM), lambda i: (0, i))]
        + [pl.BlockSpec((BM, DH), lambda i: (i, 0))] * 6,
        out_shape=[jax.ShapeDtypeStruct((NHIST, NPAD), jnp.float32)]
        + [jax.ShapeDtypeStruct((NPAD, DH), jnp.float32)] * 6,
    )(p, xa_p, xw_p)


def _tcb_body(atl, ath, awl, awh, arl, arh, sc_ref, w1t, wcat, bt, bwc,
              ha_ref, hw_ref):
    scl = sc_ref[...]
    at = jnp.concatenate([atl[...], ath[...]], axis=1) * scl[1][:, None]
    ha = jnp.dot(at, w1t[...], preferred_element_type=jnp.float32) + bt[...]
    ha_ref[...] = jnp.maximum(ha, 0.0)
    aw = jnp.concatenate([awl[...], awh[...]], axis=1) * scl[3][:, None]
    ar = jnp.concatenate([arl[...], arh[...]], axis=1) * scl[5][:, None]
    awr = jnp.concatenate([aw, ar], axis=1)
    hw = jnp.dot(awr, wcat[...], preferred_element_type=jnp.float32) + bwc[...]
    hw_ref[...] = jnp.maximum(hw, 0.0)


def _tcb(atl, ath, awl, awh, arl, arh, scales, w1t, wcat, bt, bwc):
    g = NPAD // BM
    return pl.pallas_call(
        _tcb_body,
        grid=(g,),
        in_specs=[pl.BlockSpec((BM, DH), lambda i: (i, 0))] * 6
        + [
            pl.BlockSpec((NHIST, BM), lambda i: (0, i)),
            pl.BlockSpec((D, H), lambda i: (0, 0)),
            pl.BlockSpec((2 * D, H), lambda i: (0, 0)),
            pl.BlockSpec((1, H), lambda i: (0, 0)),
            pl.BlockSpec((1, H), lambda i: (0, 0)),
        ],
        out_specs=[pl.BlockSpec((BM, H), lambda i: (i, 0))] * 2,
        out_shape=[jax.ShapeDtypeStruct((NPAD, H), jnp.float32)] * 2,
    )(atl, ath, awl, awh, arl, arh, scales, w1t, wcat, bt, bwc)


def _tcc_body(ha_ref, hw_ref, sc_ref, w2t, w2w, w2r,
              ttl, tth, twl, twh, trl, trh):
    scl = sc_ref[...]
    ha = ha_ref[...]
    hw = hw_ref[...]
    tt = jnp.dot(ha * scl[0][:, None], w2t[...], preferred_element_type=jnp.float32)
    tw = jnp.dot(hw * scl[2][:, None], w2w[...], preferred_element_type=jnp.float32)
    tr = jnp.dot(ha * scl[4][:, None], w2r[...], preferred_element_type=jnp.float32)
    ttl[...] = tt[:, :DH]
    tth[...] = tt[:, DH:]
    twl[...] = tw[:, :DH]
    twh[...] = tw[:, DH:]
    trl[...] = tr[:, :DH]
    trh[...] = tr[:, DH:]


def _tcc(ha, hw, scales, w2t, w2w, w2r):
    g = NPAD // BM
    return pl.pallas_call(
        _tcc_body,
        grid=(g,),
        in_specs=[
            pl.BlockSpec((BM, H), lambda i: (i, 0)),
            pl.BlockSpec((BM, H), lambda i: (i, 0)),
            pl.BlockSpec((NHIST, BM), lambda i: (0, i)),
            pl.BlockSpec((H, O), lambda i: (0, 0)),
            pl.BlockSpec((H, O), lambda i: (0, 0)),
            pl.BlockSpec((H, O), lambda i: (0, 0)),
        ],
        out_specs=[pl.BlockSpec((BM, DH), lambda i: (i, 0))] * 6,
        out_shape=[jax.ShapeDtypeStruct((NPAD, DH), jnp.float32)] * 6,
    )(ha, hw, scales, w2t, w2w, w2r)


def _tcd_body(utl, uth, uwl, uwh, url, urh, sc_ref, b2t, b2w, b2r,
              oa_ref, ow_ref):
    scl = sc_ref[...]
    ut = jnp.concatenate([utl[...], uth[...]], axis=1)
    uw = jnp.concatenate([uwl[...], uwh[...]], axis=1)
    ur = jnp.concatenate([url[...], urh[...]], axis=1)
    oa_ref[...] = ut * scl[1][:, None] + b2t[...]
    ow_ref[...] = 0.5 * (uw * scl[3][:, None] + b2w[...]
                         + ur * scl[5][:, None] + b2r[...])


def _tcd(utl, uth, uwl, uwh, url, urh, scales, b2t, b2w, b2r):
    g = NPAD // BM
    return pl.pallas_call(
        _tcd_body,
        grid=(g,),
        in_specs=[pl.BlockSpec((BM, DH), lambda i: (i, 0))] * 6
        + [
            pl.BlockSpec((NHIST, BM), lambda i: (0, i)),
            pl.BlockSpec((1, O), lambda i: (0, 0)),
            pl.BlockSpec((1, O), lambda i: (0, 0)),
            pl.BlockSpec((1, O), lambda i: (0, 0)),
        ],
        out_specs=[pl.BlockSpec((BM, O), lambda i: (i, 0))] * 2,
        out_shape=[jax.ShapeDtypeStruct((NPAD, O), jnp.float32)] * 2,
    )(utl, uth, uwl, uwh, url, urh, scales, b2t, b2w, b2r)


def kernel(x_acoustic, x_word, edge_sim_tic, edge_sim_w, edge_related_to,
           W1_tic, b1_tic, W1_w, b1_w, W1_rel, b1_rel,
           W2_tic, b2_tic, W2_w, b2_w, W2_rel, b2_rel):
    f32 = jnp.float32
    pad_e = lambda a: jnp.pad(a.astype(jnp.int32), (0, EPAD - E),
                              constant_values=GARBAGE)
    st, dt = pad_e(edge_sim_tic[0]), pad_e(edge_sim_tic[1])
    sw, dw = pad_e(edge_sim_w[0]), pad_e(edge_sim_w[1])
    sr, dr = pad_e(edge_related_to[0]), pad_e(edge_related_to[1])
    idx6 = jnp.concatenate([st, dt, sw, dw, sr, dr])
    to2d = lambda a: a.reshape(EPAD // 128, 128)
    xa_p = jnp.pad(x_acoustic, ((0, NPAD - N), (0, 0)))
    xw_p = jnp.pad(x_word, ((0, NPAD - N), (0, 0)))
    z = jnp.zeros((640, DH), f32)
    zh = jnp.zeros((NPAD,), f32)

    hist = _make_hist()
    agg = _make_agg()

    p = hist(idx6, zh).reshape(32, NHIST, NPAD)
    scales, xtl, xth, xwl, xwh, xrl, xrh = _tca(p, xa_p, xw_p)

    atl, ath = agg(xtl, xth, to2d(st), to2d(dt), z)
    awl, awh = agg(xwl, xwh, to2d(sw), to2d(dw), z)
    arl, arh = agg(xrl, xrh, to2d(sr), to2d(dr), z)

    wcat = jnp.concatenate([W1_w, W1_rel], axis=0) * 0.5
    bwc = (0.5 * (b1_w + b1_rel)).reshape(1, H)
    ha, hw = _tcb(atl, ath, awl, awh, arl, arh, scales,
                  W1_tic, wcat, b1_tic.reshape(1, H), bwc)

    ttl, tth, twl, twh, trl, trh = _tcc(ha, hw, scales, W2_tic, W2_w, W2_rel)

    utl, uth = agg(ttl, tth, to2d(st), to2d(dt), z)
    uwl, uwh = agg(twl, twh, to2d(sw), to2d(dw), z)
    url, urh = agg(trl, trh, to2d(sr), to2d(dr), z)

    oa, ow = _tcd(utl, uth, uwl, uwh, url, urh, scales,
                  b2_tic.reshape(1, O), b2_w.reshape(1, O), b2_r := b2_rel.reshape(1, O))
    return (oa[:N], ow[:N])


# speedup vs baseline: 3.2945x; 1.2189x over previous
import functools

import jax
import jax.numpy as jnp
from jax import lax
from jax.experimental import pallas as pl
from jax.experimental.pallas import tpu as pltpu
from jax.experimental.pallas import tpu_sc as plsc

N = 10000
NPAD = 10240
GARBAGE = 10000
E = 160000
EPAD = 163840
D = 256
DH = 128
H = 512
O = 256
BM = 1024
NHIST = 6


def _hist_body(idx_hbm, zh_hbm, out_hbm, ibuf, hist, sem):
    c = lax.axis_index("c")
    s = lax.axis_index("s")
    wid = c * 16 + s
    ones = jnp.full((16,), 1.0, jnp.float32)
    for a in range(NHIST):
        pltpu.sync_copy(zh_hbm, hist.at[pl.ds(a * NPAD, NPAD)])
    for a in range(NHIST):
        base = a * EPAD + wid * 5120

        def chunk(r, base=base, a=a):
            pltpu.sync_copy(idx_hbm.at[pl.ds(base + r * 1024, 1024)], ibuf)
            for m in range(64):
                iv = ibuf[pl.ds(m * 16, 16)] + a * NPAD
                plsc.addupdate_scatter(hist, [iv], ones)

        pl.loop(0, 5)(chunk)
    pltpu.sync_copy(hist, out_hbm.at[pl.ds(wid * (NHIST * NPAD), NHIST * NPAD)])


def _make_hist():
    mesh = plsc.VectorSubcoreMesh(core_axis_name="c", subcore_axis_name="s")
    return functools.partial(
        pl.kernel,
        mesh=mesh,
        compiler_params=pltpu.CompilerParams(needs_layout_passes=False),
        out_type=jax.ShapeDtypeStruct((32 * NHIST * NPAD,), jnp.float32),
        scratch_types=[
            pltpu.VMEM((1024,), jnp.int32),
            pltpu.VMEM((NHIST * NPAD,), jnp.float32),
            pltpu.SemaphoreType.DMA,
        ],
    )(_hist_body)


def _agg_body(vlo_hbm, vhi_hbm, src_hbm, dst_hbm, z_hbm, olo_hbm, ohi_hbm,
              sidxA, didxA, sidxB, didxB, rows, acc, gsem, ssem, isem):
    c = lax.axis_index("c")
    s = lax.axis_index("s")
    pltpu.sync_copy(z_hbm, acc.at[pl.ds(s * 640, 640)])
    plsc.subcore_barrier()

    def edge_loop(v_hbm):
        base = s * 80
        A = (sidxA, didxA)
        B = (sidxB, didxB)
        buf = lambda b: rows.at[pl.ds(b * 128, 128)]

        def gath(ib, r, b):
            pltpu.async_copy(v_hbm.at[ib.at[r]], buf(b), gsem)

        def scat(ib, r, b):
            pltpu.async_copy(buf(b), acc.at[ib.at[r]], ssem, add=True)

        def wait1(sem):
            pltpu.make_async_copy(v_hbm.at[pl.ds(0, 128)], buf(0), sem).wait()

        def waiti():
            pltpu.make_async_copy(src_hbm.at[pl.ds(0, 8)], sidxA, isem).wait()

        def chunk_body(cur, nxt, pf_start, first=False, last=False):
            for t in range(8):
                if t < 7:
                    if not (first and t == 0):
                        wait1(ssem)
                    gath(cur[0], t + 1, (t + 1) % 2)
                elif not last:
                    wait1(ssem)
                    waiti()
                    waiti()
                    gath(nxt[0], 0, 0)
                if t == 2 and not last:
                    pltpu.async_copy(src_hbm.at[pl.ds(pf_start, 8)], nxt[0], isem)
                    pltpu.async_copy(dst_hbm.at[pl.ds(pf_start, 8)], nxt[1], isem)
                wait1(gsem)
                scat(cur[1], t, t % 2)

        pltpu.sync_copy(src_hbm.at[pl.ds(base, 8)], sidxA)
        pltpu.sync_copy(dst_hbm.at[pl.ds(base, 8)], didxA)
        gath(sidxA, 0, 0)
        chunk_body(A, B, base + 8, first=True)
        chunk_body(B, A, base + 16)

        def pair(p):
            chunk_body(A, B, base + (2 * p + 1) * 8)
            chunk_body(B, A, base + (2 * p + 2) * 8)

        pl.loop(1, 4)(pair)
        chunk_body(A, B, base + 72)
        chunk_body(B, None, None, last=True)
        wait1(ssem)
        wait1(ssem)

    @pl.when(c == 0)
    def _():
        edge_loop(vlo_hbm)

    @pl.when(c == 1)
    def _():
        edge_loop(vhi_hbm)

    plsc.subcore_barrier()

    @pl.when(c == 0)
    def _():
        pltpu.sync_copy(acc.at[pl.ds(s * 640, 640)], olo_hbm.at[pl.ds(s * 640, 640)])

    @pl.when(c == 1)
    def _():
        pltpu.sync_copy(acc.at[pl.ds(s * 640, 640)], ohi_hbm.at[pl.ds(s * 640, 640)])


def _make_agg():
    mesh = plsc.VectorSubcoreMesh(core_axis_name="c", subcore_axis_name="s")
    return functools.partial(
        pl.kernel,
        mesh=mesh,
        compiler_params=pltpu.CompilerParams(needs_layout_passes=False),
        out_type=[
            jax.ShapeDtypeStruct((NPAD, DH), jnp.float32),
            jax.ShapeDtypeStruct((NPAD, DH), jnp.float32),
        ],
        scratch_types=[
            pltpu.VMEM((8, 128), jnp.int32),
            pltpu.VMEM((8, 128), jnp.int32),
            pltpu.VMEM((8, 128), jnp.int32),
            pltpu.VMEM((8, 128), jnp.int32),
            pltpu.VMEM((256, DH), jnp.float32),
            pltpu.VMEM_SHARED((NPAD, DH), jnp.float32),
            pltpu.SemaphoreType.DMA,
            pltpu.SemaphoreType.DMA,
            pltpu.SemaphoreType.DMA,
        ],
    )(_agg_body)


def _tca_body(p_ref, xa_ref, xw_ref, sc_ref, xtl, xth, xwl, xwh, xrl, xrh):
    deg = jnp.sum(p_ref[...], axis=0)
    scl = lax.rsqrt(jnp.maximum(deg, 1.0))
    sc_ref[...] = scl
    xst = xa_ref[...] * scl[0][:, None]
    xsw = xw_ref[...] * scl[2][:, None]
    xsr = xa_ref[...] * scl[4][:, None]
    xtl[...] = xst[:, :DH]
    xth[...] = xst[:, DH:]
    xwl[...] = xsw[:, :DH]
    xwh[...] = xsw[:, DH:]
    xrl[...] = xsr[:, :DH]
    xrh[...] = xsr[:, DH:]


def _tca(p, xa_p, xw_p):
    g = NPAD // BM
    return pl.pallas_call(
        _tca_body,
        grid=(g,),
        in_specs=[
            pl.BlockSpec((32, NHIST, BM), lambda i: (0, 0, i)),
            pl.BlockSpec((BM, D), lambda i: (i, 0)),
            pl.BlockSpec((BM, D), lambda i: (i, 0)),
        ],
        out_specs=[pl.BlockSpec((NHIST, BM), lambda i: (0, i))]
        + [pl.BlockSpec((BM, DH), lambda i: (i, 0))] * 6,
        out_shape=[jax.ShapeDtypeStruct((NHIST, NPAD), jnp.float32)]
        + [jax.ShapeDtypeStruct((NPAD, DH), jnp.float32)] * 6,
    )(p, xa_p, xw_p)


def _tcb_body(atl, ath, awl, awh, arl, arh, sc_ref, w1t, wcat, bt, bwc,
              ha_ref, hw_ref):
    scl = sc_ref[...]
    at = jnp.concatenate([atl[...], ath[...]], axis=1) * scl[1][:, None]
    ha = jnp.dot(at, w1t[...], preferred_element_type=jnp.float32) + bt[...]
    ha_ref[...] = jnp.maximum(ha, 0.0)
    aw = jnp.concatenate([awl[...], awh[...]], axis=1) * scl[3][:, None]
    ar = jnp.concatenate([arl[...], arh[...]], axis=1) * scl[5][:, None]
    awr = jnp.concatenate([aw, ar], axis=1)
    hw = jnp.dot(awr, wcat[...], preferred_element_type=jnp.float32) + bwc[...]
    hw_ref[...] = jnp.maximum(hw, 0.0)


def _tcb(atl, ath, awl, awh, arl, arh, scales, w1t, wcat, bt, bwc):
    g = NPAD // BM
    return pl.pallas_call(
        _tcb_body,
        grid=(g,),
        in_specs=[pl.BlockSpec((BM, DH), lambda i: (i, 0))] * 6
        + [
            pl.BlockSpec((NHIST, BM), lambda i: (0, i)),
            pl.BlockSpec((D, H), lambda i: (0, 0)),
            pl.BlockSpec((2 * D, H), lambda i: (0, 0)),
            pl.BlockSpec((1, H), lambda i: (0, 0)),
            pl.BlockSpec((1, H), lambda i: (0, 0)),
        ],
        out_specs=[pl.BlockSpec((BM, H), lambda i: (i, 0))] * 2,
        out_shape=[jax.ShapeDtypeStruct((NPAD, H), jnp.float32)] * 2,
    )(atl, ath, awl, awh, arl, arh, scales, w1t, wcat, bt, bwc)


def _tcc_body(ha_ref, hw_ref, sc_ref, w2t, w2w, w2r,
              ttl, tth, twl, twh, trl, trh):
    scl = sc_ref[...]
    ha = ha_ref[...]
    hw = hw_ref[...]
    tt = jnp.dot(ha * scl[0][:, None], w2t[...], preferred_element_type=jnp.float32)
    tw = jnp.dot(hw * scl[2][:, None], w2w[...], preferred_element_type=jnp.float32)
    tr = jnp.dot(ha * scl[4][:, None], w2r[...], preferred_element_type=jnp.float32)
    ttl[...] = tt[:, :DH]
    tth[...] = tt[:, DH:]
    twl[...] = tw[:, :DH]
    twh[...] = tw[:, DH:]
    trl[...] = tr[:, :DH]
    trh[...] = tr[:, DH:]


def _tcc(ha, hw, scales, w2t, w2w, w2r):
    g = NPAD // BM
    return pl.pallas_call(
        _tcc_body,
        grid=(g,),
        in_specs=[
            pl.BlockSpec((BM, H), lambda i: (i, 0)),
            pl.BlockSpec((BM, H), lambda i: (i, 0)),
            pl.BlockSpec((NHIST, BM), lambda i: (0, i)),
            pl.BlockSpec((H, O), lambda i: (0, 0)),
            pl.BlockSpec((H, O), lambda i: (0, 0)),
            pl.BlockSpec((H, O), lambda i: (0, 0)),
        ],
        out_specs=[pl.BlockSpec((BM, DH), lambda i: (i, 0))] * 6,
        out_shape=[jax.ShapeDtypeStruct((NPAD, DH), jnp.float32)] * 6,
    )(ha, hw, scales, w2t, w2w, w2r)


def _tcd_body(utl, uth, uwl, uwh, url, urh, sc_ref, b2t, b2w, b2r,
              oa_ref, ow_ref):
    scl = sc_ref[...]
    ut = jnp.concatenate([utl[...], uth[...]], axis=1)
    uw = jnp.concatenate([uwl[...], uwh[...]], axis=1)
    ur = jnp.concatenate([url[...], urh[...]], axis=1)
    oa_ref[...] = ut * scl[1][:, None] + b2t[...]
    ow_ref[...] = 0.5 * (uw * scl[3][:, None] + b2w[...]
                         + ur * scl[5][:, None] + b2r[...])


def _tcd(utl, uth, uwl, uwh, url, urh, scales, b2t, b2w, b2r):
    g = NPAD // BM
    return pl.pallas_call(
        _tcd_body,
        grid=(g,),
        in_specs=[pl.BlockSpec((BM, DH), lambda i: (i, 0))] * 6
        + [
            pl.BlockSpec((NHIST, BM), lambda i: (0, i)),
            pl.BlockSpec((1, O), lambda i: (0, 0)),
            pl.BlockSpec((1, O), lambda i: (0, 0)),
            pl.BlockSpec((1, O), lambda i: (0, 0)),
        ],
        out_specs=[pl.BlockSpec((BM, O), lambda i: (i, 0))] * 2,
        out_shape=[jax.ShapeDtypeStruct((NPAD, O), jnp.float32)] * 2,
    )(utl, uth, uwl, uwh, url, urh, scales, b2t, b2w, b2r)


def kernel(x_acoustic, x_word, edge_sim_tic, edge_sim_w, edge_related_to,
           W1_tic, b1_tic, W1_w, b1_w, W1_rel, b1_rel,
           W2_tic, b2_tic, W2_w, b2_w, W2_rel, b2_rel):
    f32 = jnp.float32
    pad_e = lambda a: jnp.pad(a.astype(jnp.int32), (0, EPAD - E),
                              constant_values=GARBAGE)
    st, dt = pad_e(edge_sim_tic[0]), pad_e(edge_sim_tic[1])
    sw, dw = pad_e(edge_sim_w[0]), pad_e(edge_sim_w[1])
    sr, dr = pad_e(edge_related_to[0]), pad_e(edge_related_to[1])
    idx6 = jnp.concatenate([st, dt, sw, dw, sr, dr])
    to2d = lambda a: a.reshape(EPAD // 128, 128)
    xa_p = jnp.pad(x_acoustic, ((0, NPAD - N), (0, 0)))
    xw_p = jnp.pad(x_word, ((0, NPAD - N), (0, 0)))
    z = jnp.zeros((640, DH), f32)
    zh = jnp.zeros((NPAD,), f32)

    hist = _make_hist()
    agg = _make_agg()

    p = hist(idx6, zh).reshape(32, NHIST, NPAD)
    scales, xtl, xth, xwl, xwh, xrl, xrh = _tca(p, xa_p, xw_p)

    atl, ath = agg(xtl, xth, to2d(st), to2d(dt), z)
    awl, awh = agg(xwl, xwh, to2d(sw), to2d(dw), z)
    arl, arh = agg(xrl, xrh, to2d(sr), to2d(dr), z)

    wcat = jnp.concatenate([W1_w, W1_rel], axis=0) * 0.5
    bwc = (0.5 * (b1_w + b1_rel)).reshape(1, H)
    ha, hw = _tcb(atl, ath, awl, awh, arl, arh, scales,
                  W1_tic, wcat, b1_tic.reshape(1, H), bwc)

    ttl, tth, twl, twh, trl, trh = _tcc(ha, hw, scales, W2_tic, W2_w, W2_rel)

    utl, uth = agg(ttl, tth, to2d(st), to2d(dt), z)
    uwl, uwh = agg(twl, twh, to2d(sw), to2d(dw), z)
    url, urh = agg(trl, trh, to2d(sr), to2d(dr), z)

    oa, ow = _tcd(utl, uth, uwl, uwh, url, urh, scales,
                  b2_tic.reshape(1, O), b2_w.reshape(1, O), b2_r := b2_rel.reshape(1, O))
    return (oa[:N], ow[:N])


# speedup vs baseline: 3.5927x; 1.0905x over previous
import functools

import jax
import jax.numpy as jnp
from jax import lax
from jax.experimental import pallas as pl
from jax.experimental.pallas import tpu as pltpu
from jax.experimental.pallas import tpu_sc as plsc

N = 10000
NPAD = 10240
GARBAGE = 10000
E = 160000
EPAD = 163840
D = 256
DH = 128
DQ = 64
H = 512
O = 256
BM = 1024
NHIST = 6


def _hist_body(idx_hbm, zh_hbm, out_hbm, ibuf, hist, sem):
    c = lax.axis_index("c")
    s = lax.axis_index("s")
    wid = c * 16 + s
    ones = jnp.full((16,), 1.0, jnp.float32)
    for a in range(NHIST):
        pltpu.sync_copy(zh_hbm, hist.at[pl.ds(a * NPAD, NPAD)])
    for a in range(NHIST):
        base = a * EPAD + wid * 5120

        def chunk(r, base=base, a=a):
            pltpu.sync_copy(idx_hbm.at[pl.ds(base + r * 1024, 1024)], ibuf)
            for m in range(64):
                iv = ibuf[pl.ds(m * 16, 16)] + a * NPAD
                plsc.addupdate_scatter(hist, [iv], ones)

        pl.loop(0, 5)(chunk)
    pltpu.sync_copy(hist, out_hbm.at[pl.ds(wid * (NHIST * NPAD), NHIST * NPAD)])


def _make_hist():
    mesh = plsc.VectorSubcoreMesh(core_axis_name="c", subcore_axis_name="s")
    return functools.partial(
        pl.kernel,
        mesh=mesh,
        compiler_params=pltpu.CompilerParams(needs_layout_passes=False),
        out_type=jax.ShapeDtypeStruct((32 * NHIST * NPAD,), jnp.float32),
        scratch_types=[
            pltpu.VMEM((1024,), jnp.int32),
            pltpu.VMEM((NHIST * NPAD,), jnp.float32),
            pltpu.SemaphoreType.DMA,
        ],
    )(_hist_body)


def _agg_body(vt_lo, vt_hi, vw_lo, vw_hi, vr_lo, vr_hi,
              st2d, dt2d, sw2d, dw2d, sr2d, dr2d, z_hbm,
              ot_lo, ot_hi, ow_lo, ow_hi, or_lo, or_hi,
              sidxA, didxA, sidxB, didxB, rows, acc, gsem, ssem, isem):
    c = lax.axis_index("c")
    s = lax.axis_index("s")

    def edge_loop(gsrc, src_hbm, dst_hbm):
        base = s * 80
        A = (sidxA, didxA)
        B = (sidxB, didxB)
        buf = lambda b: rows.at[pl.ds(b * 128, 128)]

        def gath(ib, r, b):
            pltpu.async_copy(gsrc.at[ib.at[r]], buf(b), gsem)

        def scat(ib, r, b):
            pltpu.async_copy(buf(b), acc.at[ib.at[r]], ssem, add=True)

        def wait1(sem):
            pltpu.make_async_copy(z_hbm.at[pl.ds(0, 128)], buf(0), sem).wait()

        def waiti():
            pltpu.make_async_copy(src_hbm.at[pl.ds(0, 8)], sidxA, isem).wait()

        def chunk_body(cur, nxt, pf_start, first=False, last=False):
            for t in range(8):
                if t < 7:
                    if not (first and t == 0):
                        wait1(ssem)
                    gath(cur[0], t + 1, (t + 1) % 2)
                elif not last:
                    wait1(ssem)
                    waiti()
                    waiti()
                    gath(nxt[0], 0, 0)
                if t == 2 and not last:
                    pltpu.async_copy(src_hbm.at[pl.ds(pf_start, 8)], nxt[0], isem)
                    pltpu.async_copy(dst_hbm.at[pl.ds(pf_start, 8)], nxt[1], isem)
                wait1(gsem)
                scat(cur[1], t, t % 2)

        pltpu.sync_copy(src_hbm.at[pl.ds(base, 8)], sidxA)
        pltpu.sync_copy(dst_hbm.at[pl.ds(base, 8)], didxA)
        gath(sidxA, 0, 0)
        chunk_body(A, B, base + 8, first=True)
        chunk_body(B, A, base + 16)

        def pair(p):
            chunk_body(A, B, base + (2 * p + 1) * 8)
            chunk_body(B, A, base + (2 * p + 2) * 8)

        pl.loop(1, 4)(pair)
        chunk_body(A, B, base + 72)
        chunk_body(B, None, None, last=True)
        wait1(ssem)
        wait1(ssem)

    def rel(v_hbm, src_hbm, dst_hbm, o_hbm):
        sl = pl.ds(s * 640, 640)
        pltpu.sync_copy(z_hbm, acc.at[sl])
        plsc.subcore_barrier()
        edge_loop(v_hbm, src_hbm, dst_hbm)
        plsc.subcore_barrier()
        pltpu.sync_copy(acc.at[sl], o_hbm.at[sl])

    @pl.when(c == 0)
    def _():
        rel(vt_lo, st2d, dt2d, ot_lo)
        rel(vw_lo, sw2d, dw2d, ow_lo)
        rel(vr_lo, sr2d, dr2d, or_lo)

    @pl.when(c == 1)
    def _():
        rel(vt_hi, st2d, dt2d, ot_hi)
        rel(vw_hi, sw2d, dw2d, ow_hi)
        rel(vr_hi, sr2d, dr2d, or_hi)


def _make_agg():
    mesh = plsc.VectorSubcoreMesh(core_axis_name="c", subcore_axis_name="s")
    return functools.partial(
        pl.kernel,
        mesh=mesh,
        compiler_params=pltpu.CompilerParams(needs_layout_passes=False),
        out_type=[jax.ShapeDtypeStruct((NPAD, DH), jnp.float32)] * 6,
        scratch_types=[
            pltpu.VMEM((8, 128), jnp.int32),
            pltpu.VMEM((8, 128), jnp.int32),
            pltpu.VMEM((8, 128), jnp.int32),
            pltpu.VMEM((8, 128), jnp.int32),
            pltpu.VMEM((256, DH), jnp.float32),
            pltpu.VMEM_SHARED((NPAD, DH), jnp.float32),
            pltpu.SemaphoreType.DMA,
            pltpu.SemaphoreType.DMA,
            pltpu.SemaphoreType.DMA,
        ],
    )(_agg_body)


def _tca_body(p_ref, xa_ref, xw_ref, sc_ref, xtl, xth, xwl, xwh, xrl, xrh):
    deg = jnp.sum(p_ref[...], axis=0)
    scl = lax.rsqrt(jnp.maximum(deg, 1.0))
    sc_ref[...] = scl
    xst = xa_ref[...] * scl[0][:, None]
    xsw = xw_ref[...] * scl[2][:, None]
    xsr = xa_ref[...] * scl[4][:, None]
    xtl[...] = xst[:, :DH]
    xth[...] = xst[:, DH:]
    xwl[...] = xsw[:, :DH]
    xwh[...] = xsw[:, DH:]
    xrl[...] = xsr[:, :DH]
    xrh[...] = xsr[:, DH:]


def _tca(p, xa_p, xw_p):
    g = NPAD // BM
    return pl.pallas_call(
        _tca_body,
        grid=(g,),
        in_specs=[
            pl.BlockSpec((32, NHIST, BM), lambda i: (0, 0, i)),
            pl.BlockSpec((BM, D), lambda i: (i, 0)),
            pl.BlockSpec((BM, D), lambda i: (i, 0)),
        ],
        out_specs=[pl.BlockSpec((NHIST, BM), lambda i: (0, i))]
        + [pl.BlockSpec((BM, DH), lambda i: (i, 0))] * 6,
        out_shape=[jax.ShapeDtypeStruct((NHIST, NPAD), jnp.float32)]
        + [jax.ShapeDtypeStruct((NPAD, DH), jnp.float32)] * 6,
    )(p, xa_p, xw_p)


def _tcb_body(atl, ath, awl, awh, arl, arh, sc_ref, w1t, wcat, bt, bwc,
              ha_ref, hw_ref):
    scl = sc_ref[...]
    at = jnp.concatenate([atl[...], ath[...]], axis=1) * scl[1][:, None]
    ha = jnp.dot(at, w1t[...], preferred_element_type=jnp.float32) + bt[...]
    ha_ref[...] = jnp.maximum(ha, 0.0)
    aw = jnp.concatenate([awl[...], awh[...]], axis=1) * scl[3][:, None]
    ar = jnp.concatenate([arl[...], arh[...]], axis=1) * scl[5][:, None]
    awr = jnp.concatenate([aw, ar], axis=1)
    hw = jnp.dot(awr, wcat[...], preferred_element_type=jnp.float32) + bwc[...]
    hw_ref[...] = jnp.maximum(hw, 0.0)


def _tcb(atl, ath, awl, awh, arl, arh, scales, w1t, wcat, bt, bwc):
    g = NPAD // BM
    return pl.pallas_call(
        _tcb_body,
        grid=(g,),
        in_specs=[pl.BlockSpec((BM, DH), lambda i: (i, 0))] * 6
        + [
            pl.BlockSpec((NHIST, BM), lambda i: (0, i)),
            pl.BlockSpec((D, H), lambda i: (0, 0)),
            pl.BlockSpec((2 * D, H), lambda i: (0, 0)),
            pl.BlockSpec((1, H), lambda i: (0, 0)),
            pl.BlockSpec((1, H), lambda i: (0, 0)),
        ],
        out_specs=[pl.BlockSpec((BM, H), lambda i: (i, 0))] * 2,
        out_shape=[jax.ShapeDtypeStruct((NPAD, H), jnp.float32)] * 2,
    )(atl, ath, awl, awh, arl, arh, scales, w1t, wcat, bt, bwc)


def _tcc_body(ha_ref, hw_ref, sc_ref, w2t, w2w, w2r,
              ttl, tth, twl, twh, trl, trh):
    scl = sc_ref[...]
    ha = ha_ref[...]
    hw = hw_ref[...]
    tt = jnp.dot(ha * scl[0][:, None], w2t[...], preferred_element_type=jnp.float32)
    tw = jnp.dot(hw * scl[2][:, None], w2w[...], preferred_element_type=jnp.float32)
    tr = jnp.dot(ha * scl[4][:, None], w2r[...], preferred_element_type=jnp.float32)
    ttl[...] = tt[:, :DH]
    tth[...] = tt[:, DH:]
    twl[...] = tw[:, :DH]
    twh[...] = tw[:, DH:]
    trl[...] = tr[:, :DH]
    trh[...] = tr[:, DH:]


def _tcc(ha, hw, scales, w2t, w2w, w2r):
    g = NPAD // BM
    return pl.pallas_call(
        _tcc_body,
        grid=(g,),
        in_specs=[
            pl.BlockSpec((BM, H), lambda i: (i, 0)),
            pl.BlockSpec((BM, H), lambda i: (i, 0)),
            pl.BlockSpec((NHIST, BM), lambda i: (0, i)),
            pl.BlockSpec((H, O), lambda i: (0, 0)),
            pl.BlockSpec((H, O), lambda i: (0, 0)),
            pl.BlockSpec((H, O), lambda i: (0, 0)),
        ],
        out_specs=[pl.BlockSpec((BM, DH), lambda i: (i, 0))] * 6,
        out_shape=[jax.ShapeDtypeStruct((NPAD, DH), jnp.float32)] * 6,
    )(ha, hw, scales, w2t, w2w, w2r)


def _tcd_body(utl, uth, uwl, uwh, url, urh, sc_ref, b2t, b2w, b2r,
              oa_ref, ow_ref):
    scl = sc_ref[...]
    ut = jnp.concatenate([utl[...], uth[...]], axis=1)
    uw = jnp.concatenate([uwl[...], uwh[...]], axis=1)
    ur = jnp.concatenate([url[...], urh[...]], axis=1)
    oa_ref[...] = ut * scl[1][:, None] + b2t[...]
    ow_ref[...] = 0.5 * (uw * scl[3][:, None] + b2w[...]
                         + ur * scl[5][:, None] + b2r[...])


def _tcd(utl, uth, uwl, uwh, url, urh, scales, b2t, b2w, b2r):
    g = NPAD // BM
    return pl.pallas_call(
        _tcd_body,
        grid=(g,),
        in_specs=[pl.BlockSpec((BM, DH), lambda i: (i, 0))] * 6
        + [
            pl.BlockSpec((NHIST, BM), lambda i: (0, i)),
            pl.BlockSpec((1, O), lambda i: (0, 0)),
            pl.BlockSpec((1, O), lambda i: (0, 0)),
            pl.BlockSpec((1, O), lambda i: (0, 0)),
        ],
        out_specs=[pl.BlockSpec((BM, O), lambda i: (i, 0))] * 2,
        out_shape=[jax.ShapeDtypeStruct((NPAD, O), jnp.float32)] * 2,
    )(utl, uth, uwl, uwh, url, urh, scales, b2t, b2w, b2r)


def kernel(x_acoustic, x_word, edge_sim_tic, edge_sim_w, edge_related_to,
           W1_tic, b1_tic, W1_w, b1_w, W1_rel, b1_rel,
           W2_tic, b2_tic, W2_w, b2_w, W2_rel, b2_rel):
    f32 = jnp.float32
    pad_e = lambda a: jnp.pad(a.astype(jnp.int32), (0, EPAD - E),
                              constant_values=GARBAGE)
    st, dt = pad_e(edge_sim_tic[0]), pad_e(edge_sim_tic[1])
    sw, dw = pad_e(edge_sim_w[0]), pad_e(edge_sim_w[1])
    sr, dr = pad_e(edge_related_to[0]), pad_e(edge_related_to[1])
    idx6 = jnp.concatenate([st, dt, sw, dw, sr, dr])
    to2d = lambda a: a.reshape(EPAD // 128, 128)
    xa_p = jnp.pad(x_acoustic, ((0, NPAD - N), (0, 0)))
    xw_p = jnp.pad(x_word, ((0, NPAD - N), (0, 0)))
    z = jnp.zeros((640, DH), f32)
    zh = jnp.zeros((NPAD,), f32)

    hist = _make_hist()
    agg = _make_agg()

    p = hist(idx6, zh).reshape(32, NHIST, NPAD)
    scales, xtl, xth, xwl, xwh, xrl, xrh = _tca(p, xa_p, xw_p)

    st2, dt2 = to2d(st), to2d(dt)
    sw2, dw2 = to2d(sw), to2d(dw)
    sr2, dr2 = to2d(sr), to2d(dr)
    atl, ath, awl, awh, arl, arh = agg(
        xtl, xth, xwl, xwh, xrl, xrh, st2, dt2, sw2, dw2, sr2, dr2, z)

    wcat = jnp.concatenate([W1_w, W1_rel], axis=0) * 0.5
    bwc = (0.5 * (b1_w + b1_rel)).reshape(1, H)
    ha, hw = _tcb(atl, ath, awl, awh, arl, arh, scales,
                  W1_tic, wcat, b1_tic.reshape(1, H), bwc)

    ttl, tth, twl, twh, trl, trh = _tcc(ha, hw, scales, W2_tic, W2_w, W2_rel)

    utl, uth, uwl, uwh, url, urh = agg(
        ttl, tth, twl, twh, trl, trh, st2, dt2, sw2, dw2, sr2, dr2, z)

    oa, ow = _tcd(utl, uth, uwl, uwh, url, urh, scales,
                  b2_tic.reshape(1, O), b2_w.reshape(1, O), b2_r := b2_rel.reshape(1, O))
    return (oa[:N], ow[:N])
